# baseline (speedup 1.0000x reference)
import jax
import jax.numpy as jnp
from jax import lax
from jax.experimental import pallas as pl
from jax.experimental.pallas import tpu as pltpu

F = 3
N = 64
P = 131072
W = 50
B = 1024

BB = 8
GRID = B // BB
LT = 128
SPILL = LT - (W + 1)


def _body(idx_ref, map_ref, cf_ref, pvm_any_ref, w_ref, pvm_alias_ref,
          x_ref, y_ref, lastw_ref, newpvm_ref,
          win, lw, sems, lw_sems, mg_sem):
    del pvm_alias_ref
    i = pl.program_id(0)
    nsteps = pl.num_programs(0)

    def start(step, slot):
        for j in range(BB):
            b = step * BB + j
            s = idx_ref[b]
            off = lax.rem(s, LT)
            a = pl.multiple_of(s - off, LT)
            pltpu.make_async_copy(
                cf_ref.at[:, :, pl.ds(a, LT)],
                win.at[slot, j, :, :, pl.ds(0, LT)],
                sems.at[slot],
            ).start()

            @pl.when(off > SPILL)
            def _():
                pltpu.make_async_copy(
                    cf_ref.at[:, :, pl.ds(pl.multiple_of(a + LT, LT), LT)],
                    win.at[slot, j, :, :, pl.ds(LT, LT)],
                    sems.at[slot],
                ).start()

            pltpu.make_async_copy(
                pvm_any_ref.at[pl.ds(s - 1, 1)],
                lw.at[slot, pl.ds(j, 1)],
                lw_sems.at[slot],
            ).start()
            pltpu.make_async_copy(
                w_ref.at[pl.ds(map_ref[b], 1)],
                newpvm_ref.at[pl.ds(s, 1)],
                mg_sem,
            ).start()

    @pl.when(i == 0)
    def _():
        start(0, 0)

    @pl.when(i + 1 < nsteps)
    def _():
        start(i + 1, (i + 1) % 2)

    slot = i % 2
    for j in range(BB):
        off = lax.rem(idx_ref[i * BB + j], LT)
        pltpu.make_async_copy(
            cf_ref.at[:, :, pl.ds(0, LT)],
            win.at[slot, j, :, :, pl.ds(0, LT)],
            sems.at[slot],
        ).wait()

        @pl.when(off > SPILL)
        def _():
            pltpu.make_async_copy(
                cf_ref.at[:, :, pl.ds(0, LT)],
                win.at[slot, j, :, :, pl.ds(LT, LT)],
                sems.at[slot],
            ).wait()

    pltpu.make_async_copy(
        pvm_any_ref.at[pl.ds(0, BB)],
        lw.at[slot],
        lw_sems.at[slot],
    ).wait()
    pltpu.make_async_copy(
        pvm_any_ref.at[pl.ds(0, BB)],
        lw.at[slot],
        mg_sem,
    ).wait()

    lane = lax.broadcasted_iota(jnp.int32, (1, 1, LT), 2)
    for j in range(BB):
        s = idx_ref[i * BB + j]
        off = lax.rem(s, LT)
        sh = lax.rem(LT - off, LT)
        t0 = win[slot, j, :, :, 0:LT]
        t1 = win[slot, j, :, :, LT:2 * LT]
        r0 = pltpu.roll(t0, sh, axis=2)
        r1 = pltpu.roll(t1, sh, axis=2)
        sel = jnp.where(lane < LT - off, r0, r1)
        inv = 1.0 / sel[0:1, :, W - 1:W]
        x_ref[j] = sel[:, :, :W] * inv
        y_ref[j] = sel[:, :, W] * inv[:, :, 0]
    lastw_ref[...] = lw[slot]


def kernel(coin_features, pvm, index, w):
    index = index.astype(jnp.int32)
    eq = index[:, None] == index[None, :]
    winner = jnp.max(
        jnp.where(eq, jnp.arange(B, dtype=jnp.int32)[None, :], -1), axis=1
    )

    grid_spec = pltpu.PrefetchScalarGridSpec(
        num_scalar_prefetch=2,
        grid=(GRID,),
        in_specs=[
            pl.BlockSpec(memory_space=pl.ANY),
            pl.BlockSpec(memory_space=pl.ANY),
            pl.BlockSpec((B, N), lambda i, *_: (0, 0)),
            pl.BlockSpec(memory_space=pl.ANY),
        ],
        out_specs=[
            pl.BlockSpec((BB, F, N, W), lambda i, *_: (i, 0, 0, 0)),
            pl.BlockSpec((BB, F, N), lambda i, *_: (i, 0, 0)),
            pl.BlockSpec((BB, N), lambda i, *_: (i, 0)),
            pl.BlockSpec(memory_space=pl.ANY),
        ],
        scratch_shapes=[
            pltpu.VMEM((2, BB, F, N, 2 * LT), jnp.float32),
            pltpu.VMEM((2, BB, N), jnp.float32),
            pltpu.SemaphoreType.DMA((2,)),
            pltpu.SemaphoreType.DMA((2,)),
            pltpu.SemaphoreType.DMA,
        ],
    )
    X, y, last_w, new_pvm = pl.pallas_call(
        _body,
        grid_spec=grid_spec,
        out_shape=[
            jax.ShapeDtypeStruct((B, F, N, W), jnp.float32),
            jax.ShapeDtypeStruct((B, F, N), jnp.float32),
            jax.ShapeDtypeStruct((B, N), jnp.float32),
            jax.ShapeDtypeStruct((P, N), jnp.float32),
        ],
        input_output_aliases={5: 3},
    )(index, winner, coin_features, pvm, w, pvm)
    return X, y, last_w, new_pvm


# speedup vs baseline: 1.0006x; 1.0006x over previous
import jax
import jax.numpy as jnp
from jax import lax
from jax.experimental import pallas as pl
from jax.experimental.pallas import tpu as pltpu

F = 3
N = 64
P = 131072
W = 50
B = 1024

BB = 8
GRID = B // BB
PCHUNK = P // GRID
LT = 128
SPILL = LT - (W + 1)
NSLOT = 4
LA = 3


def _body(idx_ref, order_ref, lrow_ref, starts_ref,
          cf_ref, pvm_any_ref, pvm_ref, w_ref,
          x_ref, y_ref, lastw_ref, newpvm_ref,
          win, lw, sems, lw_sems, mg_sem):
    i = pl.program_id(0)
    nsteps = pl.num_programs(0)

    def start(step, slot):
        for j in range(BB):
            b = step * BB + j
            s = idx_ref[b]
            off = lax.rem(s, LT)
            a = pl.multiple_of(s - off, LT)
            pltpu.make_async_copy(
                cf_ref.at[:, :, pl.ds(a, LT)],
                win.at[slot, j, :, :, pl.ds(0, LT)],
                sems.at[slot],
            ).start()

            @pl.when(off > SPILL)
            def _():
                pltpu.make_async_copy(
                    cf_ref.at[:, :, pl.ds(pl.multiple_of(a + LT, LT), LT)],
                    win.at[slot, j, :, :, pl.ds(LT, LT)],
                    sems.at[slot],
                ).start()

            pltpu.make_async_copy(
                pvm_any_ref.at[pl.ds(s - 1, 1)],
                lw.at[slot, pl.ds(j, 1)],
                lw_sems.at[slot],
            ).start()

    @pl.when(i == 0)
    def _():
        for k in range(LA):
            start(k, k)

    @pl.when(i + LA < nsteps)
    def _():
        start(i + LA, lax.rem(i + LA, NSLOT))

    newpvm_ref[...] = pvm_ref[...]

    def merge(k, carry):
        cp = pltpu.make_async_copy(
            w_ref.at[pl.ds(order_ref[k], 1)],
            newpvm_ref.at[pl.ds(lrow_ref[k], 1)],
            mg_sem,
        )
        cp.start()
        cp.wait()
        return carry

    lax.fori_loop(starts_ref[i], starts_ref[i + 1], merge, 0)

    slot = lax.rem(i, NSLOT)
    for j in range(BB):
        off = lax.rem(idx_ref[i * BB + j], LT)
        pltpu.make_async_copy(
            cf_ref.at[:, :, pl.ds(0, LT)],
            win.at[slot, j, :, :, pl.ds(0, LT)],
            sems.at[slot],
        ).wait()

        @pl.when(off > SPILL)
        def _():
            pltpu.make_async_copy(
                cf_ref.at[:, :, pl.ds(0, LT)],
                win.at[slot, j, :, :, pl.ds(LT, LT)],
                sems.at[slot],
            ).wait()

    pltpu.make_async_copy(
        pvm_any_ref.at[pl.ds(0, BB)],
        lw.at[slot],
        lw_sems.at[slot],
    ).wait()

    lane = lax.broadcasted_iota(jnp.int32, (1, 1, LT), 2)
    for j in range(BB):
        s = idx_ref[i * BB + j]
        off = lax.rem(s, LT)
        sh = lax.rem(LT - off, LT)
        t0 = win[slot, j, :, :, 0:LT]
        t1 = win[slot, j, :, :, LT:2 * LT]
        r0 = pltpu.roll(t0, sh, axis=2)
        r1 = pltpu.roll(t1, sh, axis=2)
        sel = jnp.where(lane < LT - off, r0, r1)
        inv = 1.0 / sel[0:1, :, W - 1:W]
        x_ref[j] = sel[:, :, :W] * inv
        y_ref[j] = sel[:, :, W] * inv[:, :, 0]
    lastw_ref[...] = lw[slot]


def kernel(coin_features, pvm, index, w):
    index = index.astype(jnp.int32)
    order = jnp.argsort(index, stable=True).astype(jnp.int32)
    sorted_idx = index[order]
    lrow = (sorted_idx % PCHUNK).astype(jnp.int32)
    starts = jnp.searchsorted(
        sorted_idx, jnp.arange(GRID + 1, dtype=jnp.int32) * PCHUNK
    ).astype(jnp.int32)

    grid_spec = pltpu.PrefetchScalarGridSpec(
        num_scalar_prefetch=4,
        grid=(GRID,),
        in_specs=[
            pl.BlockSpec(memory_space=pl.ANY),
            pl.BlockSpec(memory_space=pl.ANY),
            pl.BlockSpec((PCHUNK, N), lambda i, *_: (i, 0)),
            pl.BlockSpec((B, N), lambda i, *_: (0, 0)),
        ],
        out_specs=[
            pl.BlockSpec((BB, F, N, W), lambda i, *_: (i, 0, 0, 0)),
            pl.BlockSpec((BB, F, N), lambda i, *_: (i, 0, 0)),
            pl.BlockSpec((BB, N), lambda i, *_: (i, 0)),
            pl.BlockSpec((PCHUNK, N), lambda i, *_: (i, 0)),
        ],
        scratch_shapes=[
            pltpu.VMEM((NSLOT, BB, F, N, 2 * LT), jnp.float32),
            pltpu.VMEM((NSLOT, BB, N), jnp.float32),
            pltpu.SemaphoreType.DMA((NSLOT,)),
            pltpu.SemaphoreType.DMA((NSLOT,)),
            pltpu.SemaphoreType.DMA,
        ],
    )
    X, y, last_w, new_pvm = pl.pallas_call(
        _body,
        grid_spec=grid_spec,
        out_shape=[
            jax.ShapeDtypeStruct((B, F, N, W), jnp.float32),
            jax.ShapeDtypeStruct((B, F, N), jnp.float32),
            jax.ShapeDtypeStruct((B, N), jnp.float32),
            jax.ShapeDtypeStruct((P, N), jnp.float32),
        ],
    )(index, order, lrow, starts, coin_features, pvm, pvm, w)
    return X, y, last_w, new_pvm
